# baseline (speedup 1.0000x reference)
import functools

import jax
import jax.numpy as jnp
from jax import lax
from jax.experimental import pallas as pl
from jax.experimental.pallas import tpu as pltpu
from jax.experimental.pallas import tpu_sc as plsc

BATCH = 4096
N_WORDS = 50
EMB_DIM = 64
NUM_CORES = 2
NUM_SUBCORES = 16
NW = NUM_CORES * NUM_SUBCORES
CHUNK = 128
JT = EMB_DIM // 8
BT = BATCH // CHUNK

_mesh = plsc.VectorSubcoreMesh(core_axis_name="c", subcore_axis_name="s")


@functools.partial(
    pl.kernel,
    mesh=_mesh,
    out_type=jax.ShapeDtypeStruct((N_WORDS, JT, BT, 8, CHUNK), jnp.float32),
    scratch_types=[
        pltpu.VMEM((N_WORDS, CHUNK), jnp.int32),
        pltpu.VMEM((2, CHUNK, EMB_DIM), jnp.float32),
        pltpu.VMEM((2, JT, 8, CHUNK), jnp.float32),
        pltpu.SemaphoreType.DMA,
        pltpu.SemaphoreType.DMA,
        pltpu.SemaphoreType.DMA,
        pltpu.SemaphoreType.DMA,
    ],
    compiler_params=pltpu.CompilerParams(
        use_tc_tiling_on_sc=False, needs_layout_passes=False),
)
def _emb_lookup(idx_hbm, table_hbm, out_hbm, idx_v, rows_v, tiles_v,
                gsem0, gsem1, wsem0, wsem1):
    c = lax.axis_index("c")
    s = lax.axis_index("s")
    wid = s * NUM_CORES + c
    pltpu.sync_copy(idx_hbm.at[wid], idx_v)
    gsems = (gsem0, gsem1)
    wsems = (wsem0, wsem1)

    def g_desc(n, slot):
        return pltpu.make_async_copy(
            table_hbm.at[idx_v.at[n]], rows_v.at[slot], gsems[slot])

    def w_desc(n, slot, j_tile):
        return pltpu.make_async_copy(
            tiles_v.at[slot, j_tile], out_hbm.at[n, j_tile, wid], wsems[slot])

    def fire_writes(n, slot):
        for j_tile in range(JT):
            w_desc(n, slot, j_tile).start()

    def wait_writes(n, slot):
        for j_tile in range(JT):
            w_desc(n, slot, j_tile).wait()

    def build_tiles(slot):
        lanes = lax.iota(jnp.int32, 16)

        def body(j, carry):
            j_tile = j // 8
            r = j % 8
            col = jnp.full((16,), j, jnp.int32)
            for cs in range(CHUNK // 16):
                rows16 = plsc.load_gather(
                    rows_v.at[slot], [cs * 16 + lanes, col])
                tiles_v[slot, j_tile, r, pl.ds(cs * 16, 16)] = rows16
            return carry

        lax.fori_loop(0, EMB_DIM, body, 0)

    def half_step(n, slot):
        @pl.when(n + 1 < N_WORDS)
        def _():
            g_desc(n + 1, 1 - slot).start()
        g_desc(n, slot).wait()

        @pl.when(n >= 2)
        def _():
            wait_writes(n - 2, slot)
        build_tiles(slot)
        fire_writes(n, slot)

    g_desc(0, 0).start()

    def body(k, carry):
        half_step(2 * k, 0)
        half_step(2 * k + 1, 1)
        return carry

    lax.fori_loop(0, N_WORDS // 2, body, 0)
    wait_writes(N_WORDS - 2, 0)
    wait_writes(N_WORDS - 1, 1)


def kernel(input_words, embeddings):
    idx = input_words.astype(jnp.int32).reshape(NW, CHUNK, N_WORDS)
    idx = idx.transpose(0, 2, 1)
    out5 = _emb_lookup(idx, embeddings)
    out = out5.transpose(2, 4, 0, 1, 3).reshape(BATCH, N_WORDS, EMB_DIM)
    return out


# speedup vs baseline: 1.1831x; 1.1831x over previous
import functools

import jax
import jax.numpy as jnp
from jax import lax
from jax.experimental import pallas as pl
from jax.experimental.pallas import tpu as pltpu
from jax.experimental.pallas import tpu_sc as plsc

BATCH = 4096
N_WORDS = 50
EMB_DIM = 64
NUM_CORES = 2
NUM_SUBCORES = 16
NW = NUM_CORES * NUM_SUBCORES
CHUNK = 128
NCH = BATCH * N_WORDS // (NW * CHUNK)
DEPTH = 5

_mesh = plsc.VectorSubcoreMesh(core_axis_name="c", subcore_axis_name="s")


@functools.partial(
    pl.kernel,
    mesh=_mesh,
    out_type=jax.ShapeDtypeStruct((NW, NCH, CHUNK, EMB_DIM), jnp.float32),
    scratch_types=[
        pltpu.VMEM((NCH, CHUNK), jnp.int32),
        pltpu.VMEM((DEPTH, CHUNK, EMB_DIM), jnp.float32),
    ] + [pltpu.SemaphoreType.DMA] * (2 * DEPTH),
    compiler_params=pltpu.CompilerParams(
        use_tc_tiling_on_sc=False, needs_layout_passes=False),
)
def _emb_lookup(idx_hbm, table_hbm, out_hbm, idx_v, rows_v, *sems):
    c_ax = lax.axis_index("c")
    s_ax = lax.axis_index("s")
    wid = s_ax * NUM_CORES + c_ax
    pltpu.sync_copy(idx_hbm.at[wid], idx_v)
    gsems = sems[:DEPTH]
    wsems = sems[DEPTH:]

    def g_desc(c, slot):
        return pltpu.make_async_copy(
            table_hbm.at[idx_v.at[c]], rows_v.at[slot], gsems[slot])

    def w_desc(c, slot):
        return pltpu.make_async_copy(
            rows_v.at[slot], out_hbm.at[wid, c], wsems[slot])

    for c in range(DEPTH - 1):
        g_desc(c, c).start()

    def body(k, carry):
        for j in range(DEPTH):
            c = DEPTH * k + j

            @pl.when((c >= 1) & (c + DEPTH - 1 < NCH))
            def _():
                w_desc(c - 1, (j - 1) % DEPTH).wait()

            @pl.when(c + DEPTH - 1 < NCH)
            def _():
                g_desc(c + DEPTH - 1, (j - 1) % DEPTH).start()

            g_desc(c, j).wait()
            w_desc(c, j).start()
        return carry

    lax.fori_loop(0, NCH // DEPTH, body, 0)
    for s in range(DEPTH):
        w_desc(NCH - DEPTH + s, s).wait()


def kernel(input_words, embeddings):
    idx = input_words.astype(jnp.int32).reshape(NW, NCH, CHUNK)
    out = _emb_lookup(idx, embeddings)
    return out.reshape(BATCH, N_WORDS, EMB_DIM)
